# baseline (speedup 1.0000x reference)
import functools
import math

import jax
import jax.experimental.layout
import jax.numpy as jnp
from jax import lax
from jax.experimental import pallas as pl
from jax.experimental.pallas import tpu as pltpu
from jax.experimental.pallas import tpu_sc as plsc

_D = 128
_SCALE = math.sqrt(float(_D))
_NC = 2
_NS = 16
_NW = _NC * _NS
_CHUNK = 128
_NBUF = 2


def _make_lookup(total_rows: int):
    assert total_rows % (_NW * _CHUNK * _NBUF) == 0
    rows_per_w = total_rows // _NW
    n_chunks = rows_per_w // _CHUNK
    n_groups = n_chunks // _NBUF
    mesh = plsc.VectorSubcoreMesh(
        core_axis_name="c", subcore_axis_name="s", num_cores=_NC, num_subcores=_NS
    )

    @functools.partial(
        pl.kernel,
        mesh=mesh,
        out_type=jax.ShapeDtypeStruct((total_rows, _D), jnp.float32),
        scratch_types=[
            pltpu.VMEM((n_chunks, _CHUNK), jnp.int32),
            pltpu.VMEM((_CHUNK, _D), jnp.float32),
            pltpu.VMEM((_CHUNK, _D), jnp.float32),
            pltpu.VMEM((_CHUNK, _D), jnp.float32),
            pltpu.VMEM((_CHUNK, _D), jnp.float32),
            pltpu.SemaphoreType.DMA,
            pltpu.SemaphoreType.DMA,
            pltpu.SemaphoreType.DMA,
            pltpu.SemaphoreType.DMA,
        ],
    )
    def lookup(idx_hbm, table_hbm, out_hbm, idx_v, in0, in1, out0, out1,
               g0, g1, s0, s1):
        ins = (in0, in1)
        outs = (out0, out1)
        gsems = (g0, g1)
        ssems = (s0, s1)
        wid = lax.axis_index("s") * _NC + lax.axis_index("c")
        base = wid * rows_per_w
        pltpu.sync_copy(idx_hbm.at[wid], idx_v)

        def gather(c, b):
            return pltpu.make_async_copy(
                table_hbm.at[idx_v.at[c]], ins[b], gsems[b]
            )

        def store(c, b):
            return pltpu.make_async_copy(
                outs[b], out_hbm.at[pl.ds(base + c * _CHUNK, _CHUNK)], ssems[b]
            )

        for b in range(_NBUF):
            gather(b, b).start()

        def group(g, carry):
            for b in range(_NBUF):
                c = g * _NBUF + b
                gather(c, b).wait()
                @pl.when(c >= _NBUF)
                def _reclaim():
                    store(c - _NBUF, b).wait()

                def scale_row(r, _):
                    for j in range(_D // 16):
                        sl = ins[b][r, pl.ds(j * 16, 16)]
                        outs[b][r, pl.ds(j * 16, 16)] = sl * _SCALE
                    return _

                lax.fori_loop(0, _CHUNK, scale_row, 0, unroll=2)
                store(c, b).start()

                @pl.when(c + _NBUF < n_chunks)
                def _prefetch():
                    gather(c + _NBUF, b).start()

            return carry

        lax.fori_loop(0, n_groups, group, 0)
        for b in range(_NBUF):
            store(n_chunks - _NBUF + b, b).wait()

    return lookup


def _impl(x, emb_weight):
    b, l = x.shape
    total = b * l
    idx = x.reshape(_NW, total // (_NW * _CHUNK), _CHUNK).astype(jnp.int32)
    out = _make_lookup(total)(idx, emb_weight)
    return out.reshape(b, l, _D)


@functools.lru_cache(maxsize=None)
def _jitted():
    fmt = jax.experimental.layout.Format(
        jax.experimental.layout.Layout(major_to_minor=(0, 1, 2), tiling=()),
        jax.sharding.SingleDeviceSharding(jax.devices()[0]),
    )
    return jax.jit(_impl, out_shardings=fmt)


def kernel(x, emb_weight):
    return _jitted()(x, emb_weight)


# speedup vs baseline: 1.3862x; 1.3862x over previous
import functools
import math

import jax
import jax.numpy as jnp
from jax import lax
from jax.experimental import pallas as pl
from jax.experimental.pallas import tpu as pltpu
from jax.experimental.pallas import tpu_sc as plsc

_D = 128
_SCALE = math.sqrt(float(_D))
_NC = 2
_NS = 16
_NW = _NC * _NS
_NBUF = 4


def _make_lookup(B: int, L: int):
    assert B % (_NW * _NBUF) == 0
    b_per_w = B // _NW
    n_groups = b_per_w // _NBUF
    assert n_groups >= 3
    mesh = plsc.VectorSubcoreMesh(
        core_axis_name="c", subcore_axis_name="s", num_cores=_NC, num_subcores=_NS
    )

    @functools.partial(
        pl.kernel,
        mesh=mesh,
        out_type=jax.ShapeDtypeStruct((B, L, _D), jnp.float32),
        scratch_types=(
            [pltpu.VMEM((b_per_w, L), jnp.int32)]
            + [pltpu.VMEM((L, _D), jnp.float32) for _ in range(2 * _NBUF)]
            + [pltpu.SemaphoreType.DMA for _ in range(2 * _NBUF)]
        ),
        compiler_params=pltpu.CompilerParams(use_tc_tiling_on_sc=True),
    )
    def lookup(idx_hbm, table_hbm, out_hbm, idx_v, *rest):
        ins = rest[:_NBUF]
        outs = rest[_NBUF:2 * _NBUF]
        gsems = rest[2 * _NBUF:3 * _NBUF]
        ssems = rest[3 * _NBUF:4 * _NBUF]
        wid = lax.axis_index("s") * _NC + lax.axis_index("c")
        base = wid * b_per_w
        pltpu.sync_copy(idx_hbm.at[wid], idx_v)

        def gather(c, s):
            return pltpu.make_async_copy(
                table_hbm.at[idx_v.at[c]], ins[s], gsems[s]
            )

        def store(c, s):
            return pltpu.make_async_copy(outs[s], out_hbm.at[base + c], ssems[s])

        def scale(s):
            def scale_row(r, carry):
                for j in range(_D // 16):
                    outs[s][r, pl.ds(j * 16, 16)] = (
                        ins[s][r, pl.ds(j * 16, 16)] * _SCALE
                    )
                return carry

            lax.fori_loop(0, L, scale_row, 0, unroll=2)

        for s in range(_NBUF):
            gather(s, s).start()
        for s in range(_NBUF):
            gather(s, s).wait()
            scale(s)
            store(s, s).start()
            gather(s + _NBUF, s).start()

        def group(g, carry):
            for s in range(_NBUF):
                c = g * _NBUF + s
                gather(c, s).wait()
                store(c - _NBUF, s).wait()
                scale(s)
                store(c, s).start()
                gather(c + _NBUF, s).start()
            return carry

        lax.fori_loop(1, n_groups - 1, group, 0)

        for s in range(_NBUF):
            c = (n_groups - 1) * _NBUF + s
            gather(c, s).wait()
            store(c - _NBUF, s).wait()
            scale(s)
            store(c, s).start()
        for s in range(_NBUF):
            store(b_per_w - _NBUF + s, s).wait()

    return lookup


@jax.jit
def kernel(x, emb_weight):
    b, l = x.shape
    idx = x.reshape(_NW, b // _NW, l).astype(jnp.int32)
    return _make_lookup(b, l)(idx, emb_weight)
